# Initial kernel scaffold; baseline (speedup 1.0000x reference)
import jax
import jax.numpy as jnp
from jax.experimental import pallas as pl


def kernel(x, edge_index, batch, W1l, W1r, b1, W2l, W2r, b2, W3l, W3r, b3, Wlin, blin):
    raise NotImplementedError("write your pallas kernel here")


# speedup vs baseline: 2.5813x; 2.5813x over previous
import functools

import jax
import jax.numpy as jnp
from jax import lax
from jax.experimental import pallas as pl
from jax.experimental.pallas import tpu as pltpu
from jax.experimental.pallas import tpu_sc as plsc

N = 10000
E = 320000
D = 128
H = 128
C = 7
G = 64

NC = 2
NS = 16
NW = NC * NS

B_EDGE = 128
KBLK = 8
K_CHUNK = 80
E_PAD = NW * K_CHUNK * B_EDGE
ROWS_PER_SUB = 640
N_ACC = NS * ROWS_PER_SUB

_HIGH = jax.lax.Precision.HIGHEST


def _dot(a, b):
    return jnp.dot(a, b, precision=_HIGH, preferred_element_type=jnp.float32)


def _sc_agg_body(with_deg, *refs):
    if with_deg:
        (z_hbm, src_hbm, dst_hbm, out_hbm, deg_hbm,
         src_v, dst_v, rows_v, acc_sh, sem) = refs
    else:
        (z_hbm, src_hbm, dst_hbm, out_hbm,
         src_v, dst_v, rows_v, acc_sh, sem) = refs
        deg_hbm = None

    c = lax.axis_index("c")
    s = lax.axis_index("s")
    wid = c * NS + s
    base = s * ROWS_PER_SUB

    def _fill(val):
        v16 = jnp.full((16,), val, jnp.float32)

        def one(i, _):
            for k in range(H // 16):
                rows_v[i, pl.ds(k * 16, 16)] = v16
            return None

        lax.fori_loop(0, B_EDGE, one, None)

    def _zero_acc():
        _fill(0.0)
        for i in range(ROWS_PER_SUB // B_EDGE):
            pltpu.sync_copy(rows_v,
                            acc_sh.at[pl.ds(base + i * B_EDGE, B_EDGE)])

    def _writeout(dst_hbm_ref):
        pltpu.sync_copy(acc_sh.at[pl.ds(base, ROWS_PER_SUB)],
                        dst_hbm_ref.at[c, pl.ds(base, ROWS_PER_SUB)])

    _zero_acc()

    if with_deg:
        _fill(1.0)
        plsc.subcore_barrier()

        def _deg_blk(blk, _):
            pltpu.sync_copy(dst_hbm.at[wid, pl.ds(blk * KBLK, KBLK)], dst_v)
            for j in range(KBLK):
                pltpu.sync_copy(rows_v, acc_sh.at[dst_v.at[j]], add=True)
            return None

        lax.fori_loop(0, K_CHUNK // KBLK, _deg_blk, None)
        plsc.subcore_barrier()
        _writeout(deg_hbm)
        _zero_acc()

    plsc.subcore_barrier()

    def _blk(blk, _):
        pltpu.sync_copy(src_hbm.at[wid, pl.ds(blk * KBLK, KBLK)], src_v)
        pltpu.sync_copy(dst_hbm.at[wid, pl.ds(blk * KBLK, KBLK)], dst_v)
        for j in range(KBLK):
            pltpu.async_copy(z_hbm.at[src_v.at[j]], rows_v, sem).wait()
            pltpu.sync_copy(rows_v, acc_sh.at[dst_v.at[j]], add=True)
        return None

    lax.fori_loop(0, K_CHUNK // KBLK, _blk, None)

    plsc.subcore_barrier()

    _writeout(out_hbm)


def _make_sc_agg(with_deg):
    out_type = [jax.ShapeDtypeStruct((NC, N_ACC, H), jnp.float32)]
    scratch = [
        pltpu.VMEM((KBLK, B_EDGE), jnp.int32),
        pltpu.VMEM((KBLK, B_EDGE), jnp.int32),
        pltpu.VMEM((B_EDGE, H), jnp.float32),
    ]
    if with_deg:
        out_type.append(jax.ShapeDtypeStruct((NC, N_ACC, H), jnp.float32))
    scratch.append(pltpu.VMEM_SHARED((N_ACC, H), jnp.float32))
    scratch.append(pltpu.SemaphoreType.DMA)

    return pl.kernel(
        functools.partial(_sc_agg_body, with_deg),
        out_type=tuple(out_type),
        mesh=plsc.VectorSubcoreMesh(core_axis_name="c", subcore_axis_name="s"),
        scratch_types=tuple(scratch),
    )


@functools.lru_cache(maxsize=None)
def _sc_agg_cached(with_deg):
    return _make_sc_agg(with_deg)


def _sc_agg_deg(z, src_p, dst_p):
    return _sc_agg_cached(True)(z, src_p, dst_p)


def _sc_agg(z, src_p, dst_p):
    return _sc_agg_cached(False)(z, src_p, dst_p)


BN = 2000
NSTEP = N // BN

_row = pl.BlockSpec((BN, H), lambda i: (i, 0))
_row1 = pl.BlockSpec((BN, 1), lambda i: (i, 0))
_part = pl.BlockSpec((NC, BN, H), lambda i: (0, i, 0))
_full = pl.BlockSpec((H, H), lambda i: (0, 0))
_bias = pl.BlockSpec((1, H), lambda i: (0, 0))


def _tc_pre_body(x_ref, wl_ref, wr_ref, b_ref, z_ref, r_ref):
    x = x_ref[...]
    z_ref[...] = _dot(x, wl_ref[...])
    r_ref[...] = _dot(x, wr_ref[...]) + b_ref[...]


def _tc_pre(x, wl, wr, b):
    return pl.pallas_call(
        _tc_pre_body,
        grid=(NSTEP,),
        in_specs=[_row, _full, _full, _bias],
        out_specs=[_row, _row],
        out_shape=[jax.ShapeDtypeStruct((N, H), jnp.float32),
                   jax.ShapeDtypeStruct((N, H), jnp.float32)],
    )(x, wl, wr, b.reshape(1, H))


def _tc_mid_body(p_ref, degp_ref, r_ref, wl_ref, wr_ref, b_ref,
                 z_ref, rn_ref, dinv_ref):
    ssum = p_ref[0] + p_ref[1]
    deg = degp_ref[0, :, 0:1] + degp_ref[1, :, 0:1]
    dinv = 1.0 / jnp.maximum(deg, 1.0)
    dinv_ref[...] = dinv
    h = jax.nn.relu(ssum * dinv + r_ref[...])
    z_ref[...] = _dot(h, wl_ref[...])
    rn_ref[...] = _dot(h, wr_ref[...]) + b_ref[...]


def _tc_mid(p, degp, r, wl, wr, b):
    return pl.pallas_call(
        _tc_mid_body,
        grid=(NSTEP,),
        in_specs=[_part, _part, _row, _full, _full, _bias],
        out_specs=[_row, _row, _row1],
        out_shape=[jax.ShapeDtypeStruct((N, H), jnp.float32),
                   jax.ShapeDtypeStruct((N, H), jnp.float32),
                   jax.ShapeDtypeStruct((N, 1), jnp.float32)],
    )(p[:, :N, :], degp[:, :N, :], r, wl, wr, b.reshape(1, H))


def _tc_mid2_body(p_ref, dinv_ref, r_ref, wl_ref, wr_ref, b_ref,
                  z_ref, rn_ref):
    ssum = p_ref[0] + p_ref[1]
    h = jax.nn.relu(ssum * dinv_ref[...] + r_ref[...])
    z_ref[...] = _dot(h, wl_ref[...])
    rn_ref[...] = _dot(h, wr_ref[...]) + b_ref[...]


def _tc_mid2(p, dinv, r, wl, wr, b):
    return pl.pallas_call(
        _tc_mid2_body,
        grid=(NSTEP,),
        in_specs=[_part, _row1, _row, _full, _full, _bias],
        out_specs=[_row, _row],
        out_shape=[jax.ShapeDtypeStruct((N, H), jnp.float32),
                   jax.ShapeDtypeStruct((N, H), jnp.float32)],
    )(p[:, :N, :], dinv, r, wl, wr, b.reshape(1, H))


def _tc_final_body(p_ref, dinv_ref, r_ref, batch_ref, wlin_ref, blin_ref,
                   out_ref, sums_ref, counts_ref):
    i = pl.program_id(0)

    @pl.when(i == 0)
    def _init():
        sums_ref[...] = jnp.zeros_like(sums_ref)
        counts_ref[...] = jnp.zeros_like(counts_ref)

    ssum = p_ref[0] + p_ref[1]
    h = jax.nn.relu(ssum * dinv_ref[...] + r_ref[...])
    gids = lax.broadcasted_iota(jnp.int32, (G, BN), 0)
    onehot = (gids == batch_ref[0]).astype(jnp.float32)
    sums_ref[...] += _dot(onehot, h)
    counts_ref[...] += jnp.sum(onehot, axis=1, keepdims=True)

    @pl.when(i == NSTEP - 1)
    def _fin():
        pooled = sums_ref[...] / jnp.maximum(counts_ref[...], 1.0)
        logits = _dot(pooled, wlin_ref[...]) + blin_ref[...]
        m = jnp.max(logits, axis=-1, keepdims=True)
        lse = jnp.log(jnp.sum(jnp.exp(logits - m), axis=-1, keepdims=True))
        out_ref[...] = logits - m - lse


def _tc_final(p, dinv, r, batch2d, wlin, blin):
    return pl.pallas_call(
        _tc_final_body,
        grid=(NSTEP,),
        in_specs=[_part, _row1, _row,
                  pl.BlockSpec((1, 1, BN), lambda i: (i, 0, 0)),
                  pl.BlockSpec((H, C), lambda i: (0, 0)),
                  pl.BlockSpec((1, C), lambda i: (0, 0))],
        out_specs=pl.BlockSpec((G, C), lambda i: (0, 0)),
        out_shape=jax.ShapeDtypeStruct((G, C), jnp.float32),
        scratch_shapes=[pltpu.VMEM((G, H), jnp.float32),
                        pltpu.VMEM((G, 1), jnp.float32)],
    )(p[:, :N, :], dinv, r, batch2d.reshape(NSTEP, 1, BN), wlin,
      blin.reshape(1, C))


def kernel(x, edge_index, batch, W1l, W1r, b1, W2l, W2r, b2, W3l, W3r, b3,
           Wlin, blin):
    src = edge_index[0].astype(jnp.int32)
    dst = edge_index[1].astype(jnp.int32)
    pad = E_PAD - E
    src_p = jnp.concatenate([src, jnp.zeros((pad,), jnp.int32)])
    dst_p = jnp.concatenate([dst, jnp.full((pad,), N, jnp.int32)])
    src_p = src_p.reshape(NW, K_CHUNK, B_EDGE)
    dst_p = dst_p.reshape(NW, K_CHUNK, B_EDGE)
    batch2d = batch.astype(jnp.int32).reshape(1, N)

    z1, r1 = _tc_pre(x, W1l, W1r, b1)
    p1, degp = _sc_agg_deg(z1, src_p, dst_p)
    z2, r2, dinv = _tc_mid(p1, degp, r1, W2l, W2r, b2)
    (p2,) = _sc_agg(z2, src_p, dst_p)
    z3, r3 = _tc_mid2(p2, dinv, r2, W3l, W3r, b3)
    (p3,) = _sc_agg(z3, src_p, dst_p)
    return _tc_final(p3, dinv, r3, batch2d, Wlin, blin)


# speedup vs baseline: 2.8079x; 1.0878x over previous
import functools

import jax
import jax.numpy as jnp
from jax import lax
from jax.experimental import pallas as pl
from jax.experimental.pallas import tpu as pltpu
from jax.experimental.pallas import tpu_sc as plsc

N = 10000
E = 320000
D = 128
H = 128
C = 7
G = 64

NC = 2
NS = 16
NW = NC * NS

B_EDGE = 128
KBLK = 8
K_CHUNK = 80
E_PAD = NW * K_CHUNK * B_EDGE
ROWS_PER_SUB = 640
N_ACC = NS * ROWS_PER_SUB

_HIGH = jax.lax.Precision.HIGHEST


def _dot(a, b):
    return jnp.dot(a, b, precision=_HIGH, preferred_element_type=jnp.float32)


NB = K_CHUNK // KBLK


def _sc_agg_body(with_deg, *refs):
    (z_hbm, src_hbm, dst_hbm, out_hbm) = refs[:4]
    rest = refs[4:]
    if with_deg:
        deg_hbm = rest[0]
        rest = rest[1:]
    else:
        deg_hbm = None
    (src0, src1, src2, dst0, dst1, dst2, rows0, rows1,
     acc_sh, isem, gsem, ssem) = rest
    srcb = (src0, src1, src2)
    dstb = (dst0, dst1, dst2)
    rows = (rows0, rows1)

    c = lax.axis_index("c")
    s = lax.axis_index("s")
    wid = c * NS + s
    base = s * ROWS_PER_SUB

    def _fill(ref, val):
        v16 = jnp.full((16,), val, jnp.float32)

        def one(i, _):
            for k in range(H // 16):
                ref[i, pl.ds(k * 16, 16)] = v16
            return None

        lax.fori_loop(0, B_EDGE, one, None)

    def _zero_acc():
        _fill(rows0, 0.0)
        for i in range(ROWS_PER_SUB // B_EDGE):
            pltpu.sync_copy(rows0,
                            acc_sh.at[pl.ds(base + i * B_EDGE, B_EDGE)])

    def _writeout(dst_hbm_ref):
        pltpu.sync_copy(acc_sh.at[pl.ds(base, ROWS_PER_SUB)],
                        dst_hbm_ref.at[c, pl.ds(base, ROWS_PER_SUB)])

    def _stage_idx(b, sync=False):
        pr = b % 3
        if sync:
            pltpu.sync_copy(src_hbm.at[wid, pl.ds(b * KBLK, KBLK)], srcb[pr])
            pltpu.sync_copy(dst_hbm.at[wid, pl.ds(b * KBLK, KBLK)], dstb[pr])
            return None
        d1 = pltpu.async_copy(src_hbm.at[wid, pl.ds(b * KBLK, KBLK)],
                              srcb[pr], isem)
        d2 = pltpu.async_copy(dst_hbm.at[wid, pl.ds(b * KBLK, KBLK)],
                              dstb[pr], isem)
        return (d1, d2)

    def _dst_ref(j):
        return dstb[(j // KBLK) % 3].at[j % KBLK]

    def _src_ref(j):
        return srcb[(j // KBLK) % 3].at[j % KBLK]

    _zero_acc()

    if with_deg:
        _fill(rows1, 1.0)
        plsc.subcore_barrier()
        _stage_idx(0, sync=True)
        idescs = {1: _stage_idx(1)}
        sdescs = []
        for j in range(K_CHUNK):
            if j % KBLK == 0 and j > 0:
                b = j // KBLK
                for d in idescs.pop(b):
                    d.wait()
                if b + 1 < NB:
                    idescs[b + 1] = _stage_idx(b + 1)
            if j >= 2:
                sdescs[j - 2].wait()
            sdescs.append(pltpu.async_copy(rows1, acc_sh.at[_dst_ref(j)],
                                           ssem, add=True))
        sdescs[K_CHUNK - 2].wait()
        sdescs[K_CHUNK - 1].wait()
        plsc.subcore_barrier()
        _writeout(deg_hbm)
        _zero_acc()

    plsc.subcore_barrier()

    _stage_idx(0, sync=True)
    idescs = {1: _stage_idx(1)}
    gdescs = [pltpu.async_copy(z_hbm.at[_src_ref(0)], rows[0], gsem)]
    sdescs = []
    for j in range(K_CHUNK):
        gdescs[j].wait()
        sdescs.append(pltpu.async_copy(rows[j % 2], acc_sh.at[_dst_ref(j)],
                                       ssem, add=True))
        if j + 1 < K_CHUNK:
            if j >= 1:
                sdescs[j - 1].wait()
            if (j + 1) % KBLK == 0:
                b = (j + 1) // KBLK
                for d in idescs.pop(b):
                    d.wait()
                if b + 1 < NB:
                    idescs[b + 1] = _stage_idx(b + 1)
            gdescs.append(pltpu.async_copy(z_hbm.at[_src_ref(j + 1)],
                                           rows[(j + 1) % 2], gsem))
    sdescs[K_CHUNK - 2].wait()
    sdescs[K_CHUNK - 1].wait()

    plsc.subcore_barrier()

    _writeout(out_hbm)


def _make_sc_agg(with_deg):
    out_type = [jax.ShapeDtypeStruct((NC, N_ACC, H), jnp.float32)]
    if with_deg:
        out_type.append(jax.ShapeDtypeStruct((NC, N_ACC, H), jnp.float32))
    scratch = (
        [pltpu.VMEM((KBLK, B_EDGE), jnp.int32)] * 3
        + [pltpu.VMEM((KBLK, B_EDGE), jnp.int32)] * 3
        + [pltpu.VMEM((B_EDGE, H), jnp.float32)] * 2
        + [pltpu.VMEM_SHARED((N_ACC, H), jnp.float32),
           pltpu.SemaphoreType.DMA,
           pltpu.SemaphoreType.DMA,
           pltpu.SemaphoreType.DMA]
    )

    return pl.kernel(
        functools.partial(_sc_agg_body, with_deg),
        out_type=tuple(out_type),
        mesh=plsc.VectorSubcoreMesh(core_axis_name="c", subcore_axis_name="s"),
        scratch_types=tuple(scratch),
    )


@functools.lru_cache(maxsize=None)
def _sc_agg_cached(with_deg):
    return _make_sc_agg(with_deg)


def _sc_agg_deg(z, src_p, dst_p):
    return _sc_agg_cached(True)(z, src_p, dst_p)


def _sc_agg(z, src_p, dst_p):
    return _sc_agg_cached(False)(z, src_p, dst_p)


BN = 2000
NSTEP = N // BN

_row = pl.BlockSpec((BN, H), lambda i: (i, 0))
_row1 = pl.BlockSpec((BN, 1), lambda i: (i, 0))
_part = pl.BlockSpec((NC, BN, H), lambda i: (0, i, 0))
_full = pl.BlockSpec((H, H), lambda i: (0, 0))
_bias = pl.BlockSpec((1, H), lambda i: (0, 0))


def _tc_pre_body(x_ref, wl_ref, wr_ref, b_ref, z_ref, r_ref):
    x = x_ref[...]
    z_ref[...] = _dot(x, wl_ref[...])
    r_ref[...] = _dot(x, wr_ref[...]) + b_ref[...]


def _tc_pre(x, wl, wr, b):
    return pl.pallas_call(
        _tc_pre_body,
        grid=(NSTEP,),
        in_specs=[_row, _full, _full, _bias],
        out_specs=[_row, _row],
        out_shape=[jax.ShapeDtypeStruct((N, H), jnp.float32),
                   jax.ShapeDtypeStruct((N, H), jnp.float32)],
    )(x, wl, wr, b.reshape(1, H))


def _tc_mid_body(p_ref, degp_ref, r_ref, wl_ref, wr_ref, b_ref,
                 z_ref, rn_ref, dinv_ref):
    ssum = p_ref[0] + p_ref[1]
    deg = degp_ref[0, :, 0:1] + degp_ref[1, :, 0:1]
    dinv = 1.0 / jnp.maximum(deg, 1.0)
    dinv_ref[...] = dinv
    h = jax.nn.relu(ssum * dinv + r_ref[...])
    z_ref[...] = _dot(h, wl_ref[...])
    rn_ref[...] = _dot(h, wr_ref[...]) + b_ref[...]


def _tc_mid(p, degp, r, wl, wr, b):
    return pl.pallas_call(
        _tc_mid_body,
        grid=(NSTEP,),
        in_specs=[_part, _part, _row, _full, _full, _bias],
        out_specs=[_row, _row, _row1],
        out_shape=[jax.ShapeDtypeStruct((N, H), jnp.float32),
                   jax.ShapeDtypeStruct((N, H), jnp.float32),
                   jax.ShapeDtypeStruct((N, 1), jnp.float32)],
    )(p[:, :N, :], degp[:, :N, :], r, wl, wr, b.reshape(1, H))


def _tc_mid2_body(p_ref, dinv_ref, r_ref, wl_ref, wr_ref, b_ref,
                  z_ref, rn_ref):
    ssum = p_ref[0] + p_ref[1]
    h = jax.nn.relu(ssum * dinv_ref[...] + r_ref[...])
    z_ref[...] = _dot(h, wl_ref[...])
    rn_ref[...] = _dot(h, wr_ref[...]) + b_ref[...]


def _tc_mid2(p, dinv, r, wl, wr, b):
    return pl.pallas_call(
        _tc_mid2_body,
        grid=(NSTEP,),
        in_specs=[_part, _row1, _row, _full, _full, _bias],
        out_specs=[_row, _row],
        out_shape=[jax.ShapeDtypeStruct((N, H), jnp.float32),
                   jax.ShapeDtypeStruct((N, H), jnp.float32)],
    )(p[:, :N, :], dinv, r, wl, wr, b.reshape(1, H))


def _tc_final_body(p_ref, dinv_ref, r_ref, batch_ref, wlin_ref, blin_ref,
                   out_ref, sums_ref, counts_ref):
    i = pl.program_id(0)

    @pl.when(i == 0)
    def _init():
        sums_ref[...] = jnp.zeros_like(sums_ref)
        counts_ref[...] = jnp.zeros_like(counts_ref)

    ssum = p_ref[0] + p_ref[1]
    h = jax.nn.relu(ssum * dinv_ref[...] + r_ref[...])
    gids = lax.broadcasted_iota(jnp.int32, (G, BN), 0)
    onehot = (gids == batch_ref[0]).astype(jnp.float32)
    sums_ref[...] += _dot(onehot, h)
    counts_ref[...] += jnp.sum(onehot, axis=1, keepdims=True)

    @pl.when(i == NSTEP - 1)
    def _fin():
        pooled = sums_ref[...] / jnp.maximum(counts_ref[...], 1.0)
        logits = _dot(pooled, wlin_ref[...]) + blin_ref[...]
        m = jnp.max(logits, axis=-1, keepdims=True)
        lse = jnp.log(jnp.sum(jnp.exp(logits - m), axis=-1, keepdims=True))
        out_ref[...] = logits - m - lse


def _tc_final(p, dinv, r, batch2d, wlin, blin):
    return pl.pallas_call(
        _tc_final_body,
        grid=(NSTEP,),
        in_specs=[_part, _row1, _row,
                  pl.BlockSpec((1, 1, BN), lambda i: (i, 0, 0)),
                  pl.BlockSpec((H, C), lambda i: (0, 0)),
                  pl.BlockSpec((1, C), lambda i: (0, 0))],
        out_specs=pl.BlockSpec((G, C), lambda i: (0, 0)),
        out_shape=jax.ShapeDtypeStruct((G, C), jnp.float32),
        scratch_shapes=[pltpu.VMEM((G, H), jnp.float32),
                        pltpu.VMEM((G, 1), jnp.float32)],
    )(p[:, :N, :], dinv, r, batch2d.reshape(NSTEP, 1, BN), wlin,
      blin.reshape(1, C))


def kernel(x, edge_index, batch, W1l, W1r, b1, W2l, W2r, b2, W3l, W3r, b3,
           Wlin, blin):
    src = edge_index[0].astype(jnp.int32)
    dst = edge_index[1].astype(jnp.int32)
    pad = E_PAD - E
    src_p = jnp.concatenate([src, jnp.zeros((pad,), jnp.int32)])
    dst_p = jnp.concatenate([dst, jnp.full((pad,), N, jnp.int32)])
    src_p = src_p.reshape(NW, K_CHUNK, B_EDGE)
    dst_p = dst_p.reshape(NW, K_CHUNK, B_EDGE)
    batch2d = batch.astype(jnp.int32).reshape(1, N)

    z1, r1 = _tc_pre(x, W1l, W1r, b1)
    p1, degp = _sc_agg_deg(z1, src_p, dst_p)
    z2, r2, dinv = _tc_mid(p1, degp, r1, W2l, W2r, b2)
    (p2,) = _sc_agg(z2, src_p, dst_p)
    z3, r3 = _tc_mid2(p2, dinv, r2, W3l, W3r, b3)
    (p3,) = _sc_agg(z3, src_p, dst_p)
    return _tc_final(p3, dinv, r3, batch2d, Wlin, blin)


# speedup vs baseline: 9.6081x; 3.4218x over previous
import functools

import jax
import jax.numpy as jnp
from jax import lax
from jax.experimental import pallas as pl
from jax.experimental.pallas import tpu as pltpu
from jax.experimental.pallas import tpu_sc as plsc

N = 10000
E = 320000
D = 128
H = 128
C = 7
G = 64

NC = 2
NS = 16
NW = NC * NS

B_EDGE = 128
KBLK = 8
K_CHUNK = 80
E_PAD = NW * K_CHUNK * B_EDGE
ROWS_PER_SUB = 640
N_ACC = NS * ROWS_PER_SUB

_HIGH = jax.lax.Precision.HIGHEST


def _dot(a, b):
    return jnp.dot(a, b, precision=_HIGH, preferred_element_type=jnp.float32)


NB = K_CHUNK // KBLK


def _sc_agg_body(with_deg, *refs):
    (z_hbm, src_hbm, dst_hbm, out_hbm) = refs[:4]
    rest = refs[4:]
    if with_deg:
        deg_hbm = rest[0]
        rest = rest[1:]
    else:
        deg_hbm = None
    (src0, src1, src2, dst0, dst1, dst2, rows0, rows1,
     acc_sh, isem, gsem, ssem) = rest
    srcb = (src0, src1, src2)
    dstb = (dst0, dst1, dst2)
    rows = (rows0, rows1)

    c = lax.axis_index("c")
    s = lax.axis_index("s")
    wid = c * NS + s
    base = s * ROWS_PER_SUB

    def _fill(ref, val):
        v16 = jnp.full((16,), val, jnp.float32)

        def one(i, _):
            for k in range(H // 16):
                ref[i, pl.ds(k * 16, 16)] = v16
            return None

        lax.fori_loop(0, B_EDGE, one, None)

    def _zero_acc():
        _fill(rows0, 0.0)
        for i in range(ROWS_PER_SUB // B_EDGE):
            pltpu.sync_copy(rows0,
                            acc_sh.at[pl.ds(base + i * B_EDGE, B_EDGE)])

    def _writeout(dst_hbm_ref):
        pltpu.sync_copy(acc_sh.at[pl.ds(base, ROWS_PER_SUB)],
                        dst_hbm_ref.at[c, pl.ds(base, ROWS_PER_SUB)])

    def _stage_idx(b, sync=False):
        pr = b % 3
        if sync:
            pltpu.sync_copy(src_hbm.at[wid, pl.ds(b * KBLK, KBLK)], srcb[pr])
            pltpu.sync_copy(dst_hbm.at[wid, pl.ds(b * KBLK, KBLK)], dstb[pr])
            return None
        d1 = pltpu.async_copy(src_hbm.at[wid, pl.ds(b * KBLK, KBLK)],
                              srcb[pr], isem)
        d2 = pltpu.async_copy(dst_hbm.at[wid, pl.ds(b * KBLK, KBLK)],
                              dstb[pr], isem)
        return (d1, d2)

    def _dst_ref(j):
        return dstb[(j // KBLK) % 3].at[j % KBLK]

    def _src_ref(j):
        return srcb[(j // KBLK) % 3].at[j % KBLK]

    _zero_acc()

    if with_deg:
        _fill(rows1, 1.0)
        plsc.subcore_barrier()
        _stage_idx(0, sync=True)
        idescs = {1: _stage_idx(1)}
        sdescs = []
        for j in range(K_CHUNK):
            if j % KBLK == 0 and j > 0:
                b = j // KBLK
                for d in idescs.pop(b):
                    d.wait()
                if b + 1 < NB:
                    idescs[b + 1] = _stage_idx(b + 1)
            if j >= 2:
                sdescs[j - 2].wait()
            sdescs.append(pltpu.async_copy(rows1, acc_sh.at[_dst_ref(j)],
                                           ssem, add=True))
        sdescs[K_CHUNK - 2].wait()
        sdescs[K_CHUNK - 1].wait()
        plsc.subcore_barrier()
        _writeout(deg_hbm)
        _zero_acc()

    plsc.subcore_barrier()

    _stage_idx(0, sync=True)
    idescs = {1: _stage_idx(1)}
    gdescs = [pltpu.async_copy(z_hbm.at[_src_ref(0)], rows[0], gsem)]
    sdescs = []
    for j in range(K_CHUNK):
        gdescs[j].wait()
        sdescs.append(pltpu.async_copy(rows[j % 2], acc_sh.at[_dst_ref(j)],
                                       ssem, add=True))
        if j + 1 < K_CHUNK:
            if j >= 1:
                sdescs[j - 1].wait()
            if (j + 1) % KBLK == 0:
                b = (j + 1) // KBLK
                for d in idescs.pop(b):
                    d.wait()
                if b + 1 < NB:
                    idescs[b + 1] = _stage_idx(b + 1)
            gdescs.append(pltpu.async_copy(z_hbm.at[_src_ref(j + 1)],
                                           rows[(j + 1) % 2], gsem))
    sdescs[K_CHUNK - 2].wait()
    sdescs[K_CHUNK - 1].wait()

    plsc.subcore_barrier()

    _writeout(out_hbm)


def _make_sc_agg(with_deg):
    out_type = [jax.ShapeDtypeStruct((NC, N_ACC, H), jnp.float32)]
    if with_deg:
        out_type.append(jax.ShapeDtypeStruct((NC, N_ACC, H), jnp.float32))
    scratch = (
        [pltpu.VMEM((KBLK, B_EDGE), jnp.int32)] * 3
        + [pltpu.VMEM((KBLK, B_EDGE), jnp.int32)] * 3
        + [pltpu.VMEM((B_EDGE, H), jnp.float32)] * 2
        + [pltpu.VMEM_SHARED((N_ACC, H), jnp.float32),
           pltpu.SemaphoreType.DMA,
           pltpu.SemaphoreType.DMA,
           pltpu.SemaphoreType.DMA]
    )

    return pl.kernel(
        functools.partial(_sc_agg_body, with_deg),
        out_type=tuple(out_type),
        mesh=plsc.VectorSubcoreMesh(core_axis_name="c", subcore_axis_name="s"),
        scratch_types=tuple(scratch),
    )


@functools.lru_cache(maxsize=None)
def _sc_agg_cached(with_deg):
    return _make_sc_agg(with_deg)


def _sc_agg_deg(z, src_p, dst_p):
    return _sc_agg_cached(True)(z, src_p, dst_p)


def _sc_agg(z, src_p, dst_p):
    return _sc_agg_cached(False)(z, src_p, dst_p)


BN = 2000
NSTEP = N // BN

_row = pl.BlockSpec((BN, H), lambda i: (i, 0))
_row1 = pl.BlockSpec((BN, 1), lambda i: (i, 0))
_part = pl.BlockSpec((NC, BN, H), lambda i: (0, i, 0))
_full = pl.BlockSpec((H, H), lambda i: (0, 0))
_bias = pl.BlockSpec((1, H), lambda i: (0, 0))


def _tc_pre_body(x_ref, wl_ref, wr_ref, b_ref, z_ref, r_ref):
    x = x_ref[...]
    z_ref[...] = _dot(x, wl_ref[...])
    r_ref[...] = _dot(x, wr_ref[...]) + b_ref[...]


def _tc_pre(x, wl, wr, b):
    return pl.pallas_call(
        _tc_pre_body,
        grid=(NSTEP,),
        in_specs=[_row, _full, _full, _bias],
        out_specs=[_row, _row],
        out_shape=[jax.ShapeDtypeStruct((N, H), jnp.float32),
                   jax.ShapeDtypeStruct((N, H), jnp.float32)],
    )(x, wl, wr, b.reshape(1, H))


def _tc_mid_body(p_ref, degp_ref, r_ref, wl_ref, wr_ref, b_ref,
                 z_ref, rn_ref, dinv_ref):
    ssum = p_ref[0] + p_ref[1]
    deg = degp_ref[0, :, 0:1] + degp_ref[1, :, 0:1]
    dinv = 1.0 / jnp.maximum(deg, 1.0)
    dinv_ref[...] = dinv
    h = jax.nn.relu(ssum * dinv + r_ref[...])
    z_ref[...] = _dot(h, wl_ref[...])
    rn_ref[...] = _dot(h, wr_ref[...]) + b_ref[...]


def _tc_mid(p, degp, r, wl, wr, b):
    return pl.pallas_call(
        _tc_mid_body,
        grid=(NSTEP,),
        in_specs=[_part, _part, _row, _full, _full, _bias],
        out_specs=[_row, _row, _row1],
        out_shape=[jax.ShapeDtypeStruct((N, H), jnp.float32),
                   jax.ShapeDtypeStruct((N, H), jnp.float32),
                   jax.ShapeDtypeStruct((N, 1), jnp.float32)],
    )(p[:, :N, :], degp[:, :N, :], r, wl, wr, b.reshape(1, H))


def _tc_mid2_body(p_ref, dinv_ref, r_ref, wl_ref, wr_ref, b_ref,
                  z_ref, rn_ref):
    ssum = p_ref[0] + p_ref[1]
    h = jax.nn.relu(ssum * dinv_ref[...] + r_ref[...])
    z_ref[...] = _dot(h, wl_ref[...])
    rn_ref[...] = _dot(h, wr_ref[...]) + b_ref[...]


def _tc_mid2(p, dinv, r, wl, wr, b):
    return pl.pallas_call(
        _tc_mid2_body,
        grid=(NSTEP,),
        in_specs=[_part, _row1, _row, _full, _full, _bias],
        out_specs=[_row, _row],
        out_shape=[jax.ShapeDtypeStruct((N, H), jnp.float32),
                   jax.ShapeDtypeStruct((N, H), jnp.float32)],
    )(p[:, :N, :], dinv, r, wl, wr, b.reshape(1, H))


def _tc_final_body(p_ref, dinv_ref, r_ref, batch_ref, wlin_ref, blin_ref,
                   out_ref, sums_ref, counts_ref):
    i = pl.program_id(0)

    @pl.when(i == 0)
    def _init():
        sums_ref[...] = jnp.zeros_like(sums_ref)
        counts_ref[...] = jnp.zeros_like(counts_ref)

    ssum = p_ref[0] + p_ref[1]
    h = jax.nn.relu(ssum * dinv_ref[...] + r_ref[...])
    gids = lax.broadcasted_iota(jnp.int32, (G, BN), 0)
    onehot = (gids == batch_ref[0]).astype(jnp.float32)
    sums_ref[...] += _dot(onehot, h)
    counts_ref[...] += jnp.sum(onehot, axis=1, keepdims=True)

    @pl.when(i == NSTEP - 1)
    def _fin():
        pooled = sums_ref[...] / jnp.maximum(counts_ref[...], 1.0)
        logits = _dot(pooled, wlin_ref[...]) + blin_ref[...]
        m = jnp.max(logits, axis=-1, keepdims=True)
        lse = jnp.log(jnp.sum(jnp.exp(logits - m), axis=-1, keepdims=True))
        out_ref[...] = logits - m - lse


def _tc_final(p, dinv, r, batch2d, wlin, blin):
    return pl.pallas_call(
        _tc_final_body,
        grid=(NSTEP,),
        in_specs=[_part, _row1, _row,
                  pl.BlockSpec((1, 1, BN), lambda i: (i, 0, 0)),
                  pl.BlockSpec((H, C), lambda i: (0, 0)),
                  pl.BlockSpec((1, C), lambda i: (0, 0))],
        out_specs=pl.BlockSpec((G, C), lambda i: (0, 0)),
        out_shape=jax.ShapeDtypeStruct((G, C), jnp.float32),
        scratch_shapes=[pltpu.VMEM((G, H), jnp.float32),
                        pltpu.VMEM((G, 1), jnp.float32)],
    )(p[:, :N, :], dinv, r, batch2d.reshape(NSTEP, 1, BN), wlin,
      blin.reshape(1, C))


def kernel(x, edge_index, batch, W1l, W1r, b1, W2l, W2r, b2, W3l, W3r, b3,
           Wlin, blin):
    src = edge_index[0].astype(jnp.int32)
    dst = edge_index[1].astype(jnp.int32)
    pad = E_PAD - E
    pad_ids = jnp.arange(pad, dtype=jnp.int32)
    src_p = jnp.concatenate([src, pad_ids % N])
    dst_p = jnp.concatenate([dst, N + pad_ids % (N_ACC - N)])
    src_p = src_p.reshape(NW, K_CHUNK, B_EDGE)
    dst_p = dst_p.reshape(NW, K_CHUNK, B_EDGE)
    batch2d = batch.astype(jnp.int32).reshape(1, N)

    z1, r1 = _tc_pre(x, W1l, W1r, b1)
    p1, degp = _sc_agg_deg(z1, src_p, dst_p)
    z2, r2, dinv = _tc_mid(p1, degp, r1, W2l, W2r, b2)
    (p2,) = _sc_agg(z2, src_p, dst_p)
    z3, r3 = _tc_mid2(p2, dinv, r2, W3l, W3r, b3)
    (p3,) = _sc_agg(z3, src_p, dst_p)
    return _tc_final(p3, dinv, r3, batch2d, Wlin, blin)


# speedup vs baseline: 11.6943x; 1.2171x over previous
import functools

import jax
import jax.numpy as jnp
from jax import lax
from jax.experimental import pallas as pl
from jax.experimental.pallas import tpu as pltpu
from jax.experimental.pallas import tpu_sc as plsc

N = 10000
E = 320000
D = 128
H = 128
C = 7
G = 64

NC = 2
NS = 16
NW = NC * NS

B_EDGE = 64
KBLK = 8
K_CHUNK = 160
NBUF = 4
E_PAD = NW * K_CHUNK * B_EDGE
ROWS_PER_SUB = 640
N_ACC = NS * ROWS_PER_SUB

_HIGH = jax.lax.Precision.HIGHEST


def _dot(a, b):
    return jnp.dot(a, b, precision=_HIGH, preferred_element_type=jnp.float32)


NB = K_CHUNK // KBLK


def _sc_agg_body(with_deg, *refs):
    (z_hbm, src_hbm, dst_hbm, out_hbm) = refs[:4]
    rest = refs[4:]
    if with_deg:
        deg_hbm = rest[0]
        rest = rest[1:]
    else:
        deg_hbm = None
    srcb = rest[0:3]
    dstb = rest[3:6]
    rows = rest[6:6 + NBUF]
    (acc_sh, isem, gsem, ssem) = rest[6 + NBUF:]
    rows0, rows1 = rows[0], rows[1]

    c = lax.axis_index("c")
    s = lax.axis_index("s")
    wid = c * NS + s
    base = s * ROWS_PER_SUB

    def _fill(ref, val):
        v16 = jnp.full((16,), val, jnp.float32)

        def one(i, _):
            for k in range(H // 16):
                ref[i, pl.ds(k * 16, 16)] = v16
            return None

        lax.fori_loop(0, B_EDGE, one, None)

    def _zero_acc():
        _fill(rows0, 0.0)
        for i in range(ROWS_PER_SUB // B_EDGE):
            pltpu.sync_copy(rows0,
                            acc_sh.at[pl.ds(base + i * B_EDGE, B_EDGE)])

    def _writeout(dst_hbm_ref):
        pltpu.sync_copy(acc_sh.at[pl.ds(base, ROWS_PER_SUB)],
                        dst_hbm_ref.at[c, pl.ds(base, ROWS_PER_SUB)])

    def _stage_idx(b, sync=False):
        pr = b % 3
        if sync:
            pltpu.sync_copy(src_hbm.at[wid, pl.ds(b * KBLK, KBLK)], srcb[pr])
            pltpu.sync_copy(dst_hbm.at[wid, pl.ds(b * KBLK, KBLK)], dstb[pr])
            return None
        d1 = pltpu.async_copy(src_hbm.at[wid, pl.ds(b * KBLK, KBLK)],
                              srcb[pr], isem)
        d2 = pltpu.async_copy(dst_hbm.at[wid, pl.ds(b * KBLK, KBLK)],
                              dstb[pr], isem)
        return (d1, d2)

    def _dst_ref(j):
        return dstb[(j // KBLK) % 3].at[j % KBLK]

    def _src_ref(j):
        return srcb[(j // KBLK) % 3].at[j % KBLK]

    _zero_acc()

    if with_deg:
        _fill(rows1, 1.0)
        plsc.subcore_barrier()
        _stage_idx(0, sync=True)
        idescs = {1: _stage_idx(1)}
        sdescs = []
        for j in range(K_CHUNK):
            if j % KBLK == 0 and j > 0:
                b = j // KBLK
                for d in idescs.pop(b):
                    d.wait()
                if b + 1 < NB:
                    idescs[b + 1] = _stage_idx(b + 1)
            if j >= NBUF:
                sdescs[j - NBUF].wait()
            sdescs.append(pltpu.async_copy(rows1, acc_sh.at[_dst_ref(j)],
                                           ssem, add=True))
        for t in range(max(0, K_CHUNK - NBUF), K_CHUNK):
            sdescs[t].wait()
        plsc.subcore_barrier()
        _writeout(deg_hbm)
        _zero_acc()

    plsc.subcore_barrier()

    _stage_idx(0, sync=True)
    idescs = {1: _stage_idx(1)}
    gdescs = []
    sdescs = []
    for j in range(min(NBUF - 1, K_CHUNK)):
        gdescs.append(pltpu.async_copy(z_hbm.at[_src_ref(j)],
                                       rows[j % NBUF], gsem))
    for j in range(K_CHUNK):
        gdescs[j].wait()
        sdescs.append(pltpu.async_copy(rows[j % NBUF], acc_sh.at[_dst_ref(j)],
                                       ssem, add=True))
        g = j + NBUF - 1
        if g < K_CHUNK:
            if g - NBUF >= 0:
                sdescs[g - NBUF].wait()
            if g % KBLK == 0:
                b = g // KBLK
                for d in idescs.pop(b):
                    d.wait()
                if b + 1 < NB:
                    idescs[b + 1] = _stage_idx(b + 1)
            gdescs.append(pltpu.async_copy(z_hbm.at[_src_ref(g)],
                                           rows[g % NBUF], gsem))
    for t in range(max(0, K_CHUNK - NBUF), K_CHUNK):
        sdescs[t].wait()


    plsc.subcore_barrier()

    _writeout(out_hbm)


def _make_sc_agg(with_deg):
    out_type = [jax.ShapeDtypeStruct((NC, N_ACC, H), jnp.float32)]
    if with_deg:
        out_type.append(jax.ShapeDtypeStruct((NC, N_ACC, H), jnp.float32))
    scratch = (
        [pltpu.VMEM((KBLK, B_EDGE), jnp.int32)] * 3
        + [pltpu.VMEM((KBLK, B_EDGE), jnp.int32)] * 3
        + [pltpu.VMEM((B_EDGE, H), jnp.float32)] * NBUF
        + [pltpu.VMEM_SHARED((N_ACC, H), jnp.float32),
           pltpu.SemaphoreType.DMA,
           pltpu.SemaphoreType.DMA,
           pltpu.SemaphoreType.DMA]
    )

    return pl.kernel(
        functools.partial(_sc_agg_body, with_deg),
        out_type=tuple(out_type),
        mesh=plsc.VectorSubcoreMesh(core_axis_name="c", subcore_axis_name="s"),
        scratch_types=tuple(scratch),
    )


@functools.lru_cache(maxsize=None)
def _sc_agg_cached(with_deg):
    return _make_sc_agg(with_deg)


def _sc_agg_deg(z, src_p, dst_p):
    return _sc_agg_cached(True)(z, src_p, dst_p)


def _sc_agg(z, src_p, dst_p):
    return _sc_agg_cached(False)(z, src_p, dst_p)


BN = 2000
NSTEP = N // BN

_row = pl.BlockSpec((BN, H), lambda i: (i, 0))
_row1 = pl.BlockSpec((BN, 1), lambda i: (i, 0))
_part = pl.BlockSpec((NC, BN, H), lambda i: (0, i, 0))
_full = pl.BlockSpec((H, H), lambda i: (0, 0))
_bias = pl.BlockSpec((1, H), lambda i: (0, 0))


def _tc_pre_body(x_ref, wl_ref, wr_ref, b_ref, z_ref, r_ref):
    x = x_ref[...]
    z_ref[...] = _dot(x, wl_ref[...])
    r_ref[...] = _dot(x, wr_ref[...]) + b_ref[...]


def _tc_pre(x, wl, wr, b):
    return pl.pallas_call(
        _tc_pre_body,
        grid=(NSTEP,),
        in_specs=[_row, _full, _full, _bias],
        out_specs=[_row, _row],
        out_shape=[jax.ShapeDtypeStruct((N, H), jnp.float32),
                   jax.ShapeDtypeStruct((N, H), jnp.float32)],
    )(x, wl, wr, b.reshape(1, H))


def _tc_mid_body(p_ref, degp_ref, r_ref, wl_ref, wr_ref, b_ref,
                 z_ref, rn_ref, dinv_ref):
    ssum = p_ref[0] + p_ref[1]
    deg = degp_ref[0, :, 0:1] + degp_ref[1, :, 0:1]
    dinv = 1.0 / jnp.maximum(deg, 1.0)
    dinv_ref[...] = dinv
    h = jax.nn.relu(ssum * dinv + r_ref[...])
    z_ref[...] = _dot(h, wl_ref[...])
    rn_ref[...] = _dot(h, wr_ref[...]) + b_ref[...]


def _tc_mid(p, degp, r, wl, wr, b):
    return pl.pallas_call(
        _tc_mid_body,
        grid=(NSTEP,),
        in_specs=[_part, _part, _row, _full, _full, _bias],
        out_specs=[_row, _row, _row1],
        out_shape=[jax.ShapeDtypeStruct((N, H), jnp.float32),
                   jax.ShapeDtypeStruct((N, H), jnp.float32),
                   jax.ShapeDtypeStruct((N, 1), jnp.float32)],
    )(p[:, :N, :], degp[:, :N, :], r, wl, wr, b.reshape(1, H))


def _tc_mid2_body(p_ref, dinv_ref, r_ref, wl_ref, wr_ref, b_ref,
                  z_ref, rn_ref):
    ssum = p_ref[0] + p_ref[1]
    h = jax.nn.relu(ssum * dinv_ref[...] + r_ref[...])
    z_ref[...] = _dot(h, wl_ref[...])
    rn_ref[...] = _dot(h, wr_ref[...]) + b_ref[...]


def _tc_mid2(p, dinv, r, wl, wr, b):
    return pl.pallas_call(
        _tc_mid2_body,
        grid=(NSTEP,),
        in_specs=[_part, _row1, _row, _full, _full, _bias],
        out_specs=[_row, _row],
        out_shape=[jax.ShapeDtypeStruct((N, H), jnp.float32),
                   jax.ShapeDtypeStruct((N, H), jnp.float32)],
    )(p[:, :N, :], dinv, r, wl, wr, b.reshape(1, H))


def _tc_final_body(p_ref, dinv_ref, r_ref, batch_ref, wlin_ref, blin_ref,
                   out_ref, sums_ref, counts_ref):
    i = pl.program_id(0)

    @pl.when(i == 0)
    def _init():
        sums_ref[...] = jnp.zeros_like(sums_ref)
        counts_ref[...] = jnp.zeros_like(counts_ref)

    ssum = p_ref[0] + p_ref[1]
    h = jax.nn.relu(ssum * dinv_ref[...] + r_ref[...])
    gids = lax.broadcasted_iota(jnp.int32, (G, BN), 0)
    onehot = (gids == batch_ref[0]).astype(jnp.float32)
    sums_ref[...] += _dot(onehot, h)
    counts_ref[...] += jnp.sum(onehot, axis=1, keepdims=True)

    @pl.when(i == NSTEP - 1)
    def _fin():
        pooled = sums_ref[...] / jnp.maximum(counts_ref[...], 1.0)
        logits = _dot(pooled, wlin_ref[...]) + blin_ref[...]
        m = jnp.max(logits, axis=-1, keepdims=True)
        lse = jnp.log(jnp.sum(jnp.exp(logits - m), axis=-1, keepdims=True))
        out_ref[...] = logits - m - lse


def _tc_final(p, dinv, r, batch2d, wlin, blin):
    return pl.pallas_call(
        _tc_final_body,
        grid=(NSTEP,),
        in_specs=[_part, _row1, _row,
                  pl.BlockSpec((1, 1, BN), lambda i: (i, 0, 0)),
                  pl.BlockSpec((H, C), lambda i: (0, 0)),
                  pl.BlockSpec((1, C), lambda i: (0, 0))],
        out_specs=pl.BlockSpec((G, C), lambda i: (0, 0)),
        out_shape=jax.ShapeDtypeStruct((G, C), jnp.float32),
        scratch_shapes=[pltpu.VMEM((G, H), jnp.float32),
                        pltpu.VMEM((G, 1), jnp.float32)],
    )(p[:, :N, :], dinv, r, batch2d.reshape(NSTEP, 1, BN), wlin,
      blin.reshape(1, C))


def kernel(x, edge_index, batch, W1l, W1r, b1, W2l, W2r, b2, W3l, W3r, b3,
           Wlin, blin):
    src = edge_index[0].astype(jnp.int32)
    dst = edge_index[1].astype(jnp.int32)
    pad = E_PAD - E
    pad_ids = jnp.arange(pad, dtype=jnp.int32)
    src_p = jnp.concatenate([src, pad_ids % N])
    dst_p = jnp.concatenate([dst, N + pad_ids % (N_ACC - N)])
    src_p = src_p.reshape(NW, K_CHUNK, B_EDGE)
    dst_p = dst_p.reshape(NW, K_CHUNK, B_EDGE)
    batch2d = batch.astype(jnp.int32).reshape(1, N)

    z1, r1 = _tc_pre(x, W1l, W1r, b1)
    p1, degp = _sc_agg_deg(z1, src_p, dst_p)
    z2, r2, dinv = _tc_mid(p1, degp, r1, W2l, W2r, b2)
    (p2,) = _sc_agg(z2, src_p, dst_p)
    z3, r3 = _tc_mid2(p2, dinv, r2, W3l, W3r, b3)
    (p3,) = _sc_agg(z3, src_p, dst_p)
    return _tc_final(p3, dinv, r3, batch2d, Wlin, blin)


# speedup vs baseline: 14.3949x; 1.2309x over previous
import functools

import jax
import jax.numpy as jnp
from jax import lax
from jax.experimental import pallas as pl
from jax.experimental.pallas import tpu as pltpu
from jax.experimental.pallas import tpu_sc as plsc

N = 10000
E = 320000
D = 128
H = 128
C = 7
G = 64

NC = 2
NS = 16
NW = NC * NS

B_EDGE = 64
KBLK = 8
K_CHUNK = 160
NBUF = 4
DLAG = 4
E_PAD = NW * K_CHUNK * B_EDGE
ROWS_PER_SUB = 640
N_ACC = NS * ROWS_PER_SUB
DEG_NROW = N_ACC // H
PAD_ROWS = N_ACC - N
DEG_PER_SUB = N_ACC // NS

_HIGH = jax.lax.Precision.HIGHEST


def _dot(a, b):
    return jnp.dot(a, b, precision=_HIGH, preferred_element_type=jnp.float32)


NB = K_CHUNK // KBLK


def _sc_agg_body(with_deg, *refs):
    nbuf = NBUF
    (z_hbm, src_hbm, dst_hbm, out_hbm) = refs[:4]
    rest = refs[4:]
    if with_deg:
        deg_hbm = rest[0]
        rest = rest[1:]
    else:
        deg_hbm = None
    srcb = rest[0:3]
    dstb = rest[3:6]
    rows = rest[6:6 + nbuf]
    rest = rest[6 + nbuf:]
    if with_deg:
        ones1, zbuf1 = rest[:2]
        rest = rest[2:]
        (acc_sh, ldeg_sh, isem, gsem, ssem, dsem) = rest
    else:
        ones1 = zbuf1 = ldeg_sh = dsem = None
        (acc_sh, isem, gsem, ssem) = rest
    rows0 = rows[0]

    c = lax.axis_index("c")
    s = lax.axis_index("s")
    wid = c * NS + s
    base = s * ROWS_PER_SUB

    def _fill(ref, val):
        v16 = jnp.full((16,), val, jnp.float32)

        def one(i, _):
            for k in range(H // 16):
                ref[i, pl.ds(k * 16, 16)] = v16
            return None

        lax.fori_loop(0, B_EDGE, one, None)

    def _zero_acc():
        _fill(rows0, 0.0)
        for i in range(ROWS_PER_SUB // B_EDGE):
            pltpu.sync_copy(rows0,
                            acc_sh.at[pl.ds(base + i * B_EDGE, B_EDGE)])

    def _writeout(dst_hbm_ref):
        pltpu.sync_copy(acc_sh.at[pl.ds(base, ROWS_PER_SUB)],
                        dst_hbm_ref.at[c, pl.ds(base, ROWS_PER_SUB)])

    def _stage_idx(b, sync=False):
        pr = b % 3
        if sync:
            pltpu.sync_copy(src_hbm.at[wid, pl.ds(b * KBLK, KBLK)], srcb[pr])
            pltpu.sync_copy(dst_hbm.at[wid, pl.ds(b * KBLK, KBLK)], dstb[pr])
            return None
        d1 = pltpu.async_copy(src_hbm.at[wid, pl.ds(b * KBLK, KBLK)],
                              srcb[pr], isem)
        d2 = pltpu.async_copy(dst_hbm.at[wid, pl.ds(b * KBLK, KBLK)],
                              dstb[pr], isem)
        return (d1, d2)

    def _dst_ref(j):
        return dstb[(j // KBLK) % 3].at[j % KBLK]

    def _src_ref(j):
        return srcb[(j // KBLK) % 3].at[j % KBLK]

    _zero_acc()

    if with_deg:
        zeros16 = jnp.zeros((16,), jnp.float32)
        ones16 = jnp.ones((16,), jnp.float32)
        for k in range(B_EDGE // 16):
            ones1[pl.ds(k * 16, 16)] = ones16
        lax.fori_loop(0, DEG_PER_SUB // 16,
                      lambda i, _: (zbuf1.__setitem__((pl.ds(i * 16, 16),),
                                                      zeros16), None)[1],
                      None)
        pltpu.sync_copy(zbuf1, ldeg_sh.at[pl.ds(s * DEG_PER_SUB,
                                                DEG_PER_SUB)])

    plsc.subcore_barrier()

    _stage_idx(0, sync=True)
    idescs = {1: _stage_idx(1)}
    gdescs = []
    sdescs = []
    ddescs = []
    for j in range(min(nbuf - 1, K_CHUNK)):
        gdescs.append(pltpu.async_copy(z_hbm.at[_src_ref(j)],
                                       rows[j % nbuf], gsem))
    for j in range(K_CHUNK):
        gdescs[j].wait()
        sdescs.append(pltpu.async_copy(rows[j % nbuf], acc_sh.at[_dst_ref(j)],
                                       ssem, add=True))
        if with_deg:
            ddescs.append(pltpu.async_copy(ones1, ldeg_sh.at[_dst_ref(j)],
                                           dsem, add=True))
            if j >= DLAG:
                ddescs[j - DLAG].wait()
        g = j + nbuf - 1
        if g < K_CHUNK:
            if g - nbuf >= 0:
                sdescs[g - nbuf].wait()
            if g % KBLK == 0:
                b = g // KBLK
                for d in idescs.pop(b):
                    d.wait()
                if b + 1 < NB:
                    idescs[b + 1] = _stage_idx(b + 1)
            gdescs.append(pltpu.async_copy(z_hbm.at[_src_ref(g)],
                                           rows[g % nbuf], gsem))
    for t in range(max(0, K_CHUNK - nbuf), K_CHUNK):
        sdescs[t].wait()
    if with_deg:
        for t in range(max(0, K_CHUNK - DLAG), K_CHUNK):
            ddescs[t].wait()

    plsc.subcore_barrier()

    _writeout(out_hbm)
    if with_deg:
        pltpu.sync_copy(ldeg_sh.at[pl.ds(s * DEG_PER_SUB, DEG_PER_SUB)],
                        deg_hbm.at[c, pl.ds(s * DEG_PER_SUB, DEG_PER_SUB)])


def _make_sc_agg(with_deg):
    out_type = [jax.ShapeDtypeStruct((NC, N_ACC, H), jnp.float32)]
    if with_deg:
        out_type.append(jax.ShapeDtypeStruct((NC, N_ACC), jnp.float32))
    scratch = (
        [pltpu.VMEM((KBLK, B_EDGE), jnp.int32)] * 3
        + [pltpu.VMEM((KBLK, B_EDGE), jnp.int32)] * 3
        + [pltpu.VMEM((B_EDGE, H), jnp.float32)] * NBUF
        + ([pltpu.VMEM((B_EDGE,), jnp.float32),
            pltpu.VMEM((DEG_PER_SUB,), jnp.float32)]
           if with_deg else [])
        + [pltpu.VMEM_SHARED((N_ACC, H), jnp.float32)]
        + ([pltpu.VMEM_SHARED((N_ACC,), jnp.float32)]
           if with_deg else [])
        + [pltpu.SemaphoreType.DMA,
           pltpu.SemaphoreType.DMA,
           pltpu.SemaphoreType.DMA]
        + ([pltpu.SemaphoreType.DMA] if with_deg else [])
    )

    return pl.kernel(
        functools.partial(_sc_agg_body, with_deg),
        out_type=tuple(out_type),
        mesh=plsc.VectorSubcoreMesh(core_axis_name="c", subcore_axis_name="s"),
        scratch_types=tuple(scratch),
    )


@functools.lru_cache(maxsize=None)
def _sc_agg_cached(with_deg):
    return _make_sc_agg(with_deg)


def _sc_agg_deg(z, src_p, dst_p):
    return _sc_agg_cached(True)(z, src_p, dst_p)


def _sc_agg(z, src_p, dst_p):
    return _sc_agg_cached(False)(z, src_p, dst_p)


BN = 2048
NSTEP = N_ACC // BN

_row = pl.BlockSpec((BN, H), lambda i: (i, 0))
_row1 = pl.BlockSpec((BN, 1), lambda i: (i, 0))
_part = pl.BlockSpec((NC, BN, H), lambda i: (0, i, 0))
_full = pl.BlockSpec((H, H), lambda i: (0, 0))
_bias = pl.BlockSpec((1, H), lambda i: (0, 0))


def _tc_pre_body(x_ref, wl_ref, wr_ref, b_ref, z_ref, r_ref):
    x = x_ref[...]
    z_ref[...] = _dot(x, wl_ref[...])
    r_ref[...] = _dot(x, wr_ref[...]) + b_ref[...]


def _tc_pre(x, wl, wr, b):
    return pl.pallas_call(
        _tc_pre_body,
        grid=(NSTEP,),
        in_specs=[_row, _full, _full, _bias],
        out_specs=[_row, _row],
        out_shape=[jax.ShapeDtypeStruct((N_ACC, H), jnp.float32),
                   jax.ShapeDtypeStruct((N_ACC, H), jnp.float32)],
    )(x, wl, wr, b.reshape(1, H))


def _tc_mid_body(p_ref, degm_ref, r_ref, wl_ref, wr_ref, b_ref,
                 z_ref, rn_ref, dinv_ref):
    ssum = p_ref[0] + p_ref[1]
    degblk = degm_ref[0] + degm_ref[1]
    rsel = (lax.broadcasted_iota(jnp.int32, (BN, BN // H), 0) // H
            == lax.broadcasted_iota(jnp.int32, (BN, BN // H), 1))
    spread = _dot(rsel.astype(jnp.float32), degblk)
    lsel = (lax.broadcasted_iota(jnp.int32, (BN, H), 1)
            == lax.broadcasted_iota(jnp.int32, (BN, H), 0) % H)
    deg = jnp.sum(spread * lsel.astype(jnp.float32), axis=1, keepdims=True)
    dinv = 1.0 / jnp.maximum(deg, 1.0)
    dinv_ref[...] = dinv
    h = jax.nn.relu(ssum * dinv + r_ref[...])
    z_ref[...] = _dot(h, wl_ref[...])
    rn_ref[...] = _dot(h, wr_ref[...]) + b_ref[...]


def _tc_mid(p, degm, r, wl, wr, b):
    return pl.pallas_call(
        _tc_mid_body,
        grid=(NSTEP,),
        in_specs=[_part,
                  pl.BlockSpec((NC, BN // H, H), lambda i: (0, i, 0)),
                  _row, _full, _full, _bias],
        out_specs=[_row, _row, _row1],
        out_shape=[jax.ShapeDtypeStruct((N_ACC, H), jnp.float32),
                   jax.ShapeDtypeStruct((N_ACC, H), jnp.float32),
                   jax.ShapeDtypeStruct((N_ACC, 1), jnp.float32)],
    )(p, degm, r, wl, wr, b.reshape(1, H))


def _tc_mid2_body(p_ref, dinv_ref, r_ref, wl_ref, wr_ref, b_ref,
                  z_ref, rn_ref):
    ssum = p_ref[0] + p_ref[1]
    h = jax.nn.relu(ssum * dinv_ref[...] + r_ref[...])
    z_ref[...] = _dot(h, wl_ref[...])
    rn_ref[...] = _dot(h, wr_ref[...]) + b_ref[...]


def _tc_mid2(p, dinv, r, wl, wr, b):
    return pl.pallas_call(
        _tc_mid2_body,
        grid=(NSTEP,),
        in_specs=[_part, _row1, _row, _full, _full, _bias],
        out_specs=[_row, _row],
        out_shape=[jax.ShapeDtypeStruct((N_ACC, H), jnp.float32),
                   jax.ShapeDtypeStruct((N_ACC, H), jnp.float32)],
    )(p, dinv, r, wl, wr, b.reshape(1, H))


def _tc_final_body(p_ref, dinv_ref, r_ref, batch_ref, wlin_ref, blin_ref,
                   out_ref, sums_ref, counts_ref):
    i = pl.program_id(0)

    @pl.when(i == 0)
    def _init():
        sums_ref[...] = jnp.zeros_like(sums_ref)
        counts_ref[...] = jnp.zeros_like(counts_ref)

    ssum = p_ref[0] + p_ref[1]
    h = jax.nn.relu(ssum * dinv_ref[...] + r_ref[...])
    gids = lax.broadcasted_iota(jnp.int32, (G, BN), 0)
    onehot = (gids == batch_ref[0]).astype(jnp.float32)
    sums_ref[...] += _dot(onehot, h)
    counts_ref[...] += jnp.sum(onehot, axis=1, keepdims=True)

    @pl.when(i == NSTEP - 1)
    def _fin():
        pooled = sums_ref[...] / jnp.maximum(counts_ref[...], 1.0)
        logits = _dot(pooled, wlin_ref[...]) + blin_ref[...]
        m = jnp.max(logits, axis=-1, keepdims=True)
        lse = jnp.log(jnp.sum(jnp.exp(logits - m), axis=-1, keepdims=True))
        out_ref[...] = logits - m - lse


def _tc_final(p, dinv, r, batch2d, wlin, blin):
    return pl.pallas_call(
        _tc_final_body,
        grid=(NSTEP,),
        in_specs=[_part, _row1, _row,
                  pl.BlockSpec((1, 1, BN), lambda i: (i, 0, 0)),
                  pl.BlockSpec((H, C), lambda i: (0, 0)),
                  pl.BlockSpec((1, C), lambda i: (0, 0))],
        out_specs=pl.BlockSpec((G, C), lambda i: (0, 0)),
        out_shape=jax.ShapeDtypeStruct((G, C), jnp.float32),
        scratch_shapes=[pltpu.VMEM((G, H), jnp.float32),
                        pltpu.VMEM((G, 1), jnp.float32)],
    )(p, dinv, r, batch2d.reshape(NSTEP, 1, BN), wlin,
      blin.reshape(1, C))


def kernel(x, edge_index, batch, W1l, W1r, b1, W2l, W2r, b2, W3l, W3r, b3,
           Wlin, blin):
    src = edge_index[0].astype(jnp.int32)
    dst = edge_index[1].astype(jnp.int32)
    pad = E_PAD - E
    pad_ids = jnp.arange(pad, dtype=jnp.int32)
    src_p = jnp.concatenate([src, pad_ids % N])
    dst_p = jnp.concatenate([dst, N + pad_ids % PAD_ROWS])
    src_p = src_p.reshape(NW, K_CHUNK, B_EDGE)
    dst_p = dst_p.reshape(NW, K_CHUNK, B_EDGE)
    batch2d = jnp.concatenate(
        [batch.astype(jnp.int32),
         jnp.full((N_ACC - N,), G, jnp.int32)]).reshape(1, N_ACC)
    x_pad = jnp.concatenate([x, jnp.zeros((N_ACC - N, D), jnp.float32)])

    z1, r1 = _tc_pre(x_pad, W1l, W1r, b1)
    p1, degf = _sc_agg_deg(z1, src_p, dst_p)
    degm = degf.reshape(NC, DEG_NROW, H)
    z2, r2, dinv = _tc_mid(p1, degm, r1, W2l, W2r, b2)
    (p2,) = _sc_agg(z2, src_p, dst_p)
    z3, r3 = _tc_mid2(p2, dinv, r2, W3l, W3r, b3)
    (p3,) = _sc_agg(z3, src_p, dst_p)
    return _tc_final(p3, dinv, r3, batch2d, Wlin, blin)
